# Initial kernel scaffold; baseline (speedup 1.0000x reference)
import jax
import jax.numpy as jnp
from jax.experimental import pallas as pl


def kernel(x, embedding):
    raise NotImplementedError("write your pallas kernel here")


# speedup vs baseline: 1.0069x; 1.0069x over previous
import jax
import jax.numpy as jnp
from jax.experimental import pallas as pl


def _body(emb_ref, out_ref):
    b, rows, d = out_ref.shape
    out_ref[...] = jnp.broadcast_to(emb_ref[...][None], (b, rows, d))


def kernel(x, embedding):
    B, S = x.shape
    D = embedding.shape[1]
    ROWS = 512
    out = pl.pallas_call(
        _body,
        grid=(S // ROWS,),
        in_specs=[pl.BlockSpec((ROWS, D), lambda i: (i, 0))],
        out_specs=pl.BlockSpec((B, ROWS, D), lambda i: (0, i, 0)),
        out_shape=jax.ShapeDtypeStruct((B, S, D), jnp.float32),
    )(embedding[:S])
    return out


# speedup vs baseline: 1.0413x; 1.0342x over previous
import jax
import jax.numpy as jnp
from jax.experimental import pallas as pl


def _body(emb_ref, out_ref):
    b, rows, d = out_ref.shape
    out_ref[...] = jnp.broadcast_to(emb_ref[...][None], (b, rows, d))


def kernel(x, embedding):
    B, S = x.shape
    D = embedding.shape[1]
    ROWS = 1024
    out = pl.pallas_call(
        _body,
        grid=(S // ROWS,),
        in_specs=[pl.BlockSpec((ROWS, D), lambda i: (i, 0))],
        out_specs=pl.BlockSpec((B, ROWS, D), lambda i: (0, i, 0)),
        out_shape=jax.ShapeDtypeStruct((B, S, D), jnp.float32),
    )(embedding[:S])
    return out
